# baseline (speedup 1.0000x reference)
import functools

import jax
import jax.numpy as jnp
from jax.experimental import pallas as pl
from jax.experimental.pallas import tpu as pltpu

TOP_K = 122
EPS = 1e-06
import numpy as np

INT_MIN = np.int32(-2147483648)


def _toprank_gem_kernel(x_ref, o_ref, *, k):
    R = x_ref.shape[0]
    v = x_ref[...].reshape(R, -1)
    L = v.shape[1]
    b = jax.lax.bitcast_convert_type(v, jnp.int32)
    s = jnp.where(b >= 0, b, jnp.bitwise_not(b) ^ INT_MIN)

    hi = (s >> 16).astype(jnp.int16)
    lo = ((s & 0xFFFF) ^ 0x8000).astype(jnp.int16)

    kf = jnp.float32(k)
    one_b = jnp.bfloat16(1.0)
    zero_b = jnp.bfloat16(0.0)

    def packed_count(mask_ones):
        acc = mask_ones[:, 0:128]
        for i in range(1, L // 128):
            acc = acc + mask_ones[:, i * 128:(i + 1) * 128]
        return jnp.sum(acc.astype(jnp.float32), axis=1, keepdims=True)

    done = jnp.zeros((R, 1), jnp.int32)
    full_eq = jnp.zeros((R, 1), jnp.int32)

    pref1 = jnp.zeros((R, 1), jnp.int32)
    for bit in range(15, -1, -1):
        cand_u = pref1 | np.int32(1 << bit)
        cand16 = (cand_u ^ 0x8000).astype(jnp.int16)
        cnt = packed_count(jnp.where(hi >= cand16, one_b, zero_b))
        hit = jnp.logical_and(done == 0, cnt == kf)
        full_eq = jnp.where(hit, ((cand_u ^ 0x8000) << 16), full_eq)
        done = jnp.where(hit, jnp.int32(1), done)
        pref1 = jnp.where(cnt >= kf, cand_u, pref1)

    hi_t = (pref1 ^ 0x8000).astype(jnp.int16)
    a = packed_count(jnp.where(hi > hi_t, one_b, zero_b))
    lo2 = jnp.where(hi == hi_t, lo, jnp.int16(-32768))

    hi_t32 = hi_t.astype(jnp.int32) << 16

    def p2_cond(carry):
        j, _, done_c, _ = carry
        return jnp.logical_and(j < 16, jnp.any(done_c == 0))

    def p2_body(carry):
        j, pref2, done_c, full_eq_c = carry
        cand_u = pref2 | jnp.left_shift(jnp.int32(1), 15 - j)
        cand16 = (cand_u ^ 0x8000).astype(jnp.int16)
        cnt = a + packed_count(jnp.where(lo2 >= cand16, one_b, zero_b))
        hit = jnp.logical_and(done_c == 0, cnt == kf)
        full_eq_c = jnp.where(hit, hi_t32 | cand_u, full_eq_c)
        done_c = jnp.where(hit, jnp.int32(1), done_c)
        pref2 = jnp.where(cnt >= kf, cand_u, pref2)
        return j + 1, pref2, done_c, full_eq_c

    _, pref2, done, full_eq = jax.lax.while_loop(
        p2_cond, p2_body,
        (jnp.int32(0), jnp.zeros((R, 1), jnp.int32), done, full_eq))

    t_min = jnp.min(jnp.where(s >= full_eq, s, jnp.int32(2147483647)),
                    axis=1, keepdims=True)
    t_s = jnp.where(done != 0, t_min, hi_t32 | pref2)
    pu = t_s ^ INT_MIN
    t_bits = jnp.where(pu < 0, t_s, jnp.bitwise_not(pu))
    t_f = jax.lax.bitcast_convert_type(t_bits, jnp.float32)

    gt = s > t_s
    cnt_gt = jnp.sum(gt.astype(jnp.int32), axis=1, keepdims=True)
    vc = jnp.maximum(v, EPS)
    f = vc * vc * vc
    sum_gt = jnp.sum(jnp.where(gt, f, 0.0), axis=1, keepdims=True)

    tc = jnp.maximum(t_f, EPS)
    ft = tc * tc * tc
    total = sum_gt + (k - cnt_gt).astype(jnp.float32) * ft
    pooled = total * (1.0 / k)
    o_ref[...] = jnp.exp(jnp.log(pooled) * (1.0 / 3.0)).reshape(o_ref.shape)


@jax.jit
def kernel(x):
    N, C, H, W = x.shape
    L = H * W
    k = TOP_K
    xf = x.reshape(N * C, H, W)
    rows = N * C
    R = 384
    grid = (rows // R,)
    out = pl.pallas_call(
        functools.partial(_toprank_gem_kernel, k=k),
        grid=grid,
        in_specs=[pl.BlockSpec((R, H, W), lambda i: (i, 0, 0))],
        out_specs=pl.BlockSpec((R, 1), lambda i: (i, 0)),
        out_shape=jax.ShapeDtypeStruct((rows, 1), jnp.float32),
        compiler_params=pltpu.CompilerParams(
            dimension_semantics=("parallel",),
        ),
    )(xf)
    return out.reshape(N, C, 1)


# speedup vs baseline: 1.0868x; 1.0868x over previous
import functools

import jax
import jax.numpy as jnp
from jax.experimental import pallas as pl
from jax.experimental.pallas import tpu as pltpu

TOP_K = 122
EPS = 1e-06
import numpy as np

INT_MIN = np.int32(-2147483648)


def _toprank_gem_kernel(x_ref, o_ref, *, k):
    R = x_ref.shape[0]
    v = x_ref[...].reshape(R, -1)
    L = v.shape[1]
    b = jax.lax.bitcast_convert_type(v, jnp.int32)
    s = jnp.where(b >= 0, b, jnp.bitwise_not(b) ^ INT_MIN)

    hi = (s >> 16).astype(jnp.int16)
    lo = ((s & 0xFFFF) ^ 0x8000).astype(jnp.int16)

    kf = jnp.float32(k)
    one_b = jnp.bfloat16(1.0)
    zero_b = jnp.bfloat16(0.0)

    def packed_count(mask_ones):
        acc = mask_ones[:, 0:128]
        for i in range(1, L // 128):
            acc = acc + mask_ones[:, i * 128:(i + 1) * 128]
        return jnp.sum(acc.astype(jnp.float32), axis=1, keepdims=True)

    pref1 = jnp.zeros((R, 1), jnp.int32)
    for bit in range(15, -1, -1):
        cand_u = pref1 | np.int32(1 << bit)
        cand16 = (cand_u ^ 0x8000).astype(jnp.int16)
        cnt = packed_count(jnp.where(hi >= cand16, one_b, zero_b))
        pref1 = jnp.where(cnt >= kf, cand_u, pref1)

    hi_t = (pref1 ^ 0x8000).astype(jnp.int16)
    a = packed_count(jnp.where(hi > hi_t, one_b, zero_b))
    lo2 = jnp.where(hi == hi_t, lo, jnp.int16(-32768))

    pref2 = jnp.zeros((R, 1), jnp.int32)
    for bit in range(15, -1, -1):
        cand_u = pref2 | np.int32(1 << bit)
        cand16 = (cand_u ^ 0x8000).astype(jnp.int16)
        cnt = a + packed_count(jnp.where(lo2 >= cand16, one_b, zero_b))
        pref2 = jnp.where(cnt >= kf, cand_u, pref2)

    t_s = (hi_t.astype(jnp.int32) << 16) | pref2
    pu = t_s ^ INT_MIN
    t_bits = jnp.where(pu < 0, t_s, jnp.bitwise_not(pu))
    t_f = jax.lax.bitcast_convert_type(t_bits, jnp.float32)

    gt = s > t_s
    cnt_gt = jnp.sum(gt.astype(jnp.int32), axis=1, keepdims=True)
    vc = jnp.maximum(v, EPS)
    f = vc * vc * vc
    sum_gt = jnp.sum(jnp.where(gt, f, 0.0), axis=1, keepdims=True)

    tc = jnp.maximum(t_f, EPS)
    ft = tc * tc * tc
    total = sum_gt + (k - cnt_gt).astype(jnp.float32) * ft
    pooled = total * (1.0 / k)
    o_ref[...] = jnp.exp(jnp.log(pooled) * (1.0 / 3.0)).reshape(o_ref.shape)


@jax.jit
def kernel(x):
    N, C, H, W = x.shape
    L = H * W
    k = TOP_K
    xf = x.reshape(N * C, H, W)
    rows = N * C
    R = 384
    grid = (rows // R,)
    out = pl.pallas_call(
        functools.partial(_toprank_gem_kernel, k=k),
        grid=grid,
        in_specs=[pl.BlockSpec((R, H, W), lambda i: (i, 0, 0))],
        out_specs=pl.BlockSpec((R, 1), lambda i: (i, 0)),
        out_shape=jax.ShapeDtypeStruct((rows, 1), jnp.float32),
        compiler_params=pltpu.CompilerParams(
            dimension_semantics=("parallel",),
        ),
    )(xf)
    return out.reshape(N, C, 1)
